# Initial kernel scaffold; baseline (speedup 1.0000x reference)
import functools

import jax
import jax.numpy as jnp
from jax import lax
from jax.experimental import pallas as pl
from jax.experimental.pallas import tpu as pltpu
from jax.experimental.pallas import tpu_sc as plsc

B = 16384
N_TOKEN_FIELDS = 26
FIELD_DIM = 40000
SEQ_LEN = 50
SEQ_PAD = 64
N_FLOAT_FIELDS = 8
EMB = 16
N_OUT = N_TOKEN_FIELDS + 1 + N_FLOAT_FIELDS

CB = 8


def _make_kernel(nw):
    b_per_w = B // nw
    n_chunks = b_per_w // CB
    mesh = plsc.VectorSubcoreMesh(core_axis_name="c", subcore_axis_name="s")

    @functools.partial(
        pl.kernel,
        mesh=mesh,
        out_type=jax.ShapeDtypeStruct((B, N_OUT, EMB), jnp.float32),
        scratch_types=[
            pltpu.VMEM((CB, N_TOKEN_FIELDS), jnp.int32),
            pltpu.VMEM((CB, SEQ_PAD), jnp.int32),
            pltpu.VMEM((CB, N_FLOAT_FIELDS), jnp.float32),
            pltpu.VMEM((N_FLOAT_FIELDS, EMB), jnp.float32),
            pltpu.VMEM((CB, SEQ_LEN, EMB), jnp.float32),
            pltpu.VMEM((CB, N_OUT, EMB), jnp.float32),
            pltpu.SemaphoreType.DMA,
        ],
    )
    def body(tok_idx_hbm, seq_idx_hbm, ff_hbm, tok_tab, seq_tab, ft_hbm,
             out_hbm, tok_idx_v, seq_idx_v, ff_v, ft_v, seq_buf, out_v, sem):
        nc = plsc.get_sparse_core_info().num_cores
        wid = lax.axis_index("s") * nc + lax.axis_index("c")

        pltpu.sync_copy(ft_hbm, ft_v)
        ft_rows = [ft_v[f] for f in range(N_FLOAT_FIELDS)]

        def chunk(c, carry):
            base = wid * b_per_w + c * CB
            pltpu.sync_copy(tok_idx_hbm.at[pl.ds(base, CB)], tok_idx_v)
            pltpu.sync_copy(seq_idx_hbm.at[pl.ds(base, CB)], seq_idx_v)
            pltpu.sync_copy(ff_hbm.at[pl.ds(base, CB)], ff_v)

            handles = []
            for i in range(CB):
                handles.append(pltpu.async_copy(
                    tok_tab.at[tok_idx_v.at[i]],
                    out_v.at[i, pl.ds(0, N_TOKEN_FIELDS)], sem))
                handles.append(pltpu.async_copy(
                    seq_tab.at[seq_idx_v.at[i, pl.ds(0, SEQ_LEN)]],
                    seq_buf.at[i], sem))
            for h in handles:
                h.wait()

            for i in range(CB):
                accs = [jnp.zeros((EMB,), jnp.float32) for _ in range(4)]
                for l in range(SEQ_LEN):
                    accs[l % 4] = accs[l % 4] + seq_buf[i, l]
                summed = (accs[0] + accs[1]) + (accs[2] + accs[3])
                cvec = jnp.zeros((16,), jnp.float32)
                for j in range(4):
                    iv = seq_idx_v[i, pl.ds(16 * j, 16)]
                    cvec = cvec + jnp.where(iv != 0, 1.0, 0.0)
                cnt = jnp.sum(cvec)
                out_v[i, N_TOKEN_FIELDS] = summed / (cnt + jnp.float32(1e-8))

                for f in range(N_FLOAT_FIELDS):
                    val = plsc.load_gather(
                        ff_v,
                        [jnp.full((16,), i, jnp.int32),
                         jnp.full((16,), f, jnp.int32)])
                    out_v[i, N_TOKEN_FIELDS + 1 + f] = ft_rows[f] * val

            pltpu.sync_copy(out_v, out_hbm.at[pl.ds(base, CB)])
            return carry

        lax.fori_loop(0, n_chunks, chunk, None)

    return body


def kernel(token_fields, token_seq_field, float_fields, token_table,
           seq_table, float_table):
    offsets = jnp.arange(N_TOKEN_FIELDS, dtype=jnp.int32) * FIELD_DIM
    shifted = (token_fields.astype(jnp.int32) + offsets[None, :])
    seq_idx = jnp.pad(token_seq_field.astype(jnp.int32),
                      ((0, 0), (0, SEQ_PAD - SEQ_LEN)))
    info = plsc.get_sparse_core_info()
    nw = info.num_cores * info.num_subcores
    return _make_kernel(nw)(shifted, seq_idx, float_fields, token_table,
                            seq_table, float_table)


# speedup vs baseline: 2.6556x; 2.6556x over previous
import functools

import jax
import jax.numpy as jnp
from jax import lax
from jax.experimental import pallas as pl
from jax.experimental.pallas import tpu as pltpu
from jax.experimental.pallas import tpu_sc as plsc

B = 16384
N_TOKEN_FIELDS = 26
FIELD_DIM = 40000
SEQ_LEN = 50
SEQ_PAD = 64
SEQ_GATHER = 56
N_FLOAT_FIELDS = 8
EMB = 16
N_OUT = N_TOKEN_FIELDS + 1 + N_FLOAT_FIELDS

CB = 16


def _make_kernel(nw):
    b_per_w = B // nw
    n_chunks = b_per_w // CB
    mesh = plsc.VectorSubcoreMesh(core_axis_name="c", subcore_axis_name="s")

    @functools.partial(
        pl.kernel,
        mesh=mesh,
        out_type=jax.ShapeDtypeStruct((B, N_OUT, EMB), jnp.float32),
        compiler_params=pltpu.CompilerParams(
            needs_layout_passes=False, use_tc_tiling_on_sc=False),
        scratch_types=[
            pltpu.VMEM((CB, N_TOKEN_FIELDS), jnp.int32),
            pltpu.VMEM((CB, SEQ_PAD), jnp.int32),
            pltpu.VMEM((CB * N_FLOAT_FIELDS,), jnp.float32),
            pltpu.VMEM((N_FLOAT_FIELDS, EMB), jnp.float32),
            pltpu.VMEM((CB, SEQ_GATHER, EMB), jnp.float32),
            pltpu.VMEM((CB, N_OUT, EMB), jnp.float32),
            pltpu.SemaphoreType.DMA,
        ],
    )
    def body(tok_idx_hbm, seq_idx_hbm, ff_hbm, tok_tab, seq_tab, ft_hbm,
             out_hbm, tok_idx_v, seq_idx_v, ff_v, ft_v, seq_buf, out_v, sem):
        nc = plsc.get_sparse_core_info().num_cores
        wid = lax.axis_index("s") * nc + lax.axis_index("c")

        pltpu.sync_copy(ft_hbm, ft_v)
        ft_rows = [ft_v[f] for f in range(N_FLOAT_FIELDS)]

        def chunk(c, carry):
            base = wid * b_per_w + c * CB
            pltpu.sync_copy(tok_idx_hbm.at[pl.ds(base, CB)], tok_idx_v)
            pltpu.sync_copy(seq_idx_hbm.at[pl.ds(base, CB)], seq_idx_v)
            pltpu.sync_copy(
                ff_hbm.at[pl.ds(base * N_FLOAT_FIELDS, CB * N_FLOAT_FIELDS)],
                ff_v)

            handles = []
            for i in range(CB):
                handles.append(pltpu.async_copy(
                    tok_tab.at[tok_idx_v.at[i]],
                    out_v.at[i, pl.ds(0, N_TOKEN_FIELDS)], sem))
                handles.append(pltpu.async_copy(
                    seq_tab.at[seq_idx_v.at[i, pl.ds(0, SEQ_GATHER)]],
                    seq_buf.at[i], sem))
            for h in handles:
                h.wait()

            lanes = jnp.arange(16, dtype=jnp.int32)
            cacc = jnp.zeros((16,), jnp.int32)
            for l in range(SEQ_LEN):
                col = plsc.load_gather(
                    seq_idx_v, [lanes, jnp.full((16,), l, jnp.int32)])
                cacc = cacc + jnp.where(col != 0, 1, 0)
            inv_vec = jnp.float32(1.0) / (
                cacc.astype(jnp.float32) + jnp.float32(1e-8))
            ffvecs = [ff_v[pl.ds(g * 16, 16)]
                      for g in range(CB * N_FLOAT_FIELDS // 16)]

            for i in range(CB):
                accs = [jnp.zeros((EMB,), jnp.float32) for _ in range(4)]
                for l in range(SEQ_LEN):
                    accs[l % 4] = accs[l % 4] + seq_buf[i, l]
                summed = (accs[0] + accs[1]) + (accs[2] + accs[3])
                out_v[i, N_TOKEN_FIELDS] = summed * inv_vec[i]

                for f in range(N_FLOAT_FIELDS):
                    p = i * N_FLOAT_FIELDS + f
                    val = ffvecs[p // 16][p % 16]
                    out_v[i, N_TOKEN_FIELDS + 1 + f] = ft_rows[f] * val

            pltpu.sync_copy(out_v, out_hbm.at[pl.ds(base, CB)])
            return carry

        lax.fori_loop(0, n_chunks, chunk, None)

    return body


def kernel(token_fields, token_seq_field, float_fields, token_table,
           seq_table, float_table):
    offsets = jnp.arange(N_TOKEN_FIELDS, dtype=jnp.int32) * FIELD_DIM
    shifted = (token_fields.astype(jnp.int32) + offsets[None, :])
    seq_idx = jnp.pad(token_seq_field.astype(jnp.int32),
                      ((0, 0), (0, SEQ_PAD - SEQ_LEN)))
    info = plsc.get_sparse_core_info()
    nw = info.num_cores * info.num_subcores
    return _make_kernel(nw)(shifted, seq_idx,
                            float_fields.reshape(B * N_FLOAT_FIELDS),
                            token_table, seq_table, float_table)


# speedup vs baseline: 2.6602x; 1.0017x over previous
import functools

import jax
import jax.numpy as jnp
from jax import lax
from jax.experimental import pallas as pl
from jax.experimental.pallas import tpu as pltpu
from jax.experimental.pallas import tpu_sc as plsc

B = 16384
N_TOKEN_FIELDS = 26
FIELD_DIM = 40000
SEQ_LEN = 50
SEQ_PAD = 64
SEQ_GATHER = 56
N_FLOAT_FIELDS = 8
EMB = 16
N_OUT = N_TOKEN_FIELDS + 1 + N_FLOAT_FIELDS

CB = 16


def _make_kernel(nw):
    b_per_w = B // nw
    n_chunks = b_per_w // CB
    mesh = plsc.VectorSubcoreMesh(core_axis_name="c", subcore_axis_name="s")

    @functools.partial(
        pl.kernel,
        mesh=mesh,
        out_type=jax.ShapeDtypeStruct((B, N_OUT, EMB), jnp.float32),
        compiler_params=pltpu.CompilerParams(
            needs_layout_passes=False, use_tc_tiling_on_sc=False),
        scratch_types=[
            pltpu.VMEM((2, CB, N_TOKEN_FIELDS), jnp.int32),
            pltpu.VMEM((2, CB, SEQ_PAD), jnp.int32),
            pltpu.VMEM((2, CB * N_FLOAT_FIELDS), jnp.float32),
            pltpu.VMEM((N_FLOAT_FIELDS, EMB), jnp.float32),
            pltpu.VMEM((2, CB, SEQ_GATHER, EMB), jnp.float32),
            pltpu.VMEM((2, CB, N_OUT, EMB), jnp.float32),
            pltpu.SemaphoreType.DMA,
            pltpu.SemaphoreType.DMA,
            pltpu.SemaphoreType.DMA,
            pltpu.SemaphoreType.DMA,
            pltpu.SemaphoreType.DMA,
            pltpu.SemaphoreType.DMA,
        ],
    )
    def body(tok_idx_hbm, seq_idx_hbm, ff_hbm, tok_tab, seq_tab, ft_hbm,
             out_hbm, tok_idx_v, seq_idx_v, ff_v, ft_v, seq_buf, out_v,
             sem_g0, sem_g1, sem_i0, sem_i1, sem_o0, sem_o1):
        nc = plsc.get_sparse_core_info().num_cores
        wid = lax.axis_index("s") * nc + lax.axis_index("c")
        sem_g = (sem_g0, sem_g1)
        sem_i = (sem_i0, sem_i1)
        sem_o = (sem_o0, sem_o1)

        def fire_idx(j, p):
            base = wid * b_per_w + j * CB
            pltpu.async_copy(tok_idx_hbm.at[pl.ds(base, CB)],
                             tok_idx_v.at[p], sem_i[p])
            pltpu.async_copy(seq_idx_hbm.at[pl.ds(base, CB)],
                             seq_idx_v.at[p], sem_i[p])
            pltpu.async_copy(
                ff_hbm.at[pl.ds(base * N_FLOAT_FIELDS, CB * N_FLOAT_FIELDS)],
                ff_v.at[p], sem_i[p])

        def wait_idx(p):
            pltpu.make_async_copy(tok_idx_hbm.at[pl.ds(0, CB)],
                                  tok_idx_v.at[p], sem_i[p]).wait()
            pltpu.make_async_copy(seq_idx_hbm.at[pl.ds(0, CB)],
                                  seq_idx_v.at[p], sem_i[p]).wait()
            pltpu.make_async_copy(ff_hbm.at[pl.ds(0, CB * N_FLOAT_FIELDS)],
                                  ff_v.at[p], sem_i[p]).wait()

        def gather_descs(p):
            for i in range(CB):
                yield pltpu.make_async_copy(
                    tok_tab.at[tok_idx_v.at[p, i]],
                    out_v.at[p, i, pl.ds(0, N_TOKEN_FIELDS)], sem_g[p])
                yield pltpu.make_async_copy(
                    seq_tab.at[seq_idx_v.at[p, i, pl.ds(0, SEQ_GATHER)]],
                    seq_buf.at[p, i], sem_g[p])

        def fire_gathers(p):
            for d in gather_descs(p):
                d.start()

        def drain_gathers(p):
            for d in gather_descs(p):
                d.wait()

        def fire_out(j, p):
            base = wid * b_per_w + j * CB
            pltpu.async_copy(out_v.at[p], out_hbm.at[pl.ds(base, CB)],
                             sem_o[p])

        def wait_out(p):
            pltpu.make_async_copy(out_v.at[p], out_hbm.at[pl.ds(0, CB)],
                                  sem_o[p]).wait()

        pltpu.sync_copy(ft_hbm, ft_v)
        ft_rows = [ft_v[f] for f in range(N_FLOAT_FIELDS)]
        lanes = jnp.arange(16, dtype=jnp.int32)

        fire_idx(0, 0)
        wait_idx(0)
        fire_gathers(0)
        fire_idx(1, 1)

        def pair(t, carry):
            for k in range(2):
                j = 2 * t + k
                p, q = k, 1 - k

                drain_gathers(p)

                cacc = jnp.zeros((16,), jnp.int32)
                for l in range(SEQ_LEN):
                    col = plsc.load_gather(
                        seq_idx_v.at[p],
                        [lanes, jnp.full((16,), l, jnp.int32)])
                    cacc = cacc + jnp.where(col != 0, 1, 0)
                inv_vec = jnp.float32(1.0) / (
                    cacc.astype(jnp.float32) + jnp.float32(1e-8))
                ffvecs = [ff_v[p, pl.ds(g * 16, 16)]
                          for g in range(CB * N_FLOAT_FIELDS // 16)]

                @pl.when(j + 2 < n_chunks)
                def _():
                    fire_idx(j + 2, p)

                @pl.when(j >= 1)
                def _():
                    wait_out(q)

                @pl.when(j + 1 < n_chunks)
                def _():
                    wait_idx(q)
                    fire_gathers(q)

                for i in range(CB):
                    accs = [jnp.zeros((EMB,), jnp.float32) for _ in range(4)]
                    for l in range(SEQ_LEN):
                        accs[l % 4] = accs[l % 4] + seq_buf[p, i, l]
                    summed = (accs[0] + accs[1]) + (accs[2] + accs[3])
                    out_v[p, i, N_TOKEN_FIELDS] = summed * inv_vec[i]
                    for f in range(N_FLOAT_FIELDS):
                        pos = i * N_FLOAT_FIELDS + f
                        val = ffvecs[pos // 16][pos % 16]
                        out_v[p, i, N_TOKEN_FIELDS + 1 + f] = ft_rows[f] * val

                fire_out(j, p)
            return carry

        lax.fori_loop(0, n_chunks // 2, pair, None)
        wait_out(1)

    return body


def kernel(token_fields, token_seq_field, float_fields, token_table,
           seq_table, float_table):
    offsets = jnp.arange(N_TOKEN_FIELDS, dtype=jnp.int32) * FIELD_DIM
    shifted = (token_fields.astype(jnp.int32) + offsets[None, :])
    seq_idx = jnp.pad(token_seq_field.astype(jnp.int32),
                      ((0, 0), (0, SEQ_PAD - SEQ_LEN)))
    info = plsc.get_sparse_core_info()
    nw = info.num_cores * info.num_subcores
    return _make_kernel(nw)(shifted, seq_idx,
                            float_fields.reshape(B * N_FLOAT_FIELDS),
                            token_table, seq_table, float_table)


# speedup vs baseline: 3.8595x; 1.4508x over previous
import functools

import jax
import jax.numpy as jnp
from jax import lax
from jax.experimental import pallas as pl
from jax.experimental.pallas import tpu as pltpu
from jax.experimental.pallas import tpu_sc as plsc

B = 16384
N_TOKEN_FIELDS = 26
FIELD_DIM = 40000
SEQ_LEN = 50
N_FLOAT_FIELDS = 8
EMB = 16
N_OUT = N_TOKEN_FIELDS + 1 + N_FLOAT_FIELDS

CB = 16
TOK_W = CB * N_TOKEN_FIELDS
SEQ_W = CB * SEQ_LEN
SEQ_SEGS = [(0, 120), (120, 120), (240, 120), (360, 120),
            (480, 120), (600, 120), (720, 80)]


def _make_kernel(nw):
    b_per_w = B // nw
    n_chunks = b_per_w // CB
    mesh = plsc.VectorSubcoreMesh(core_axis_name="c", subcore_axis_name="s")

    @functools.partial(
        pl.kernel,
        mesh=mesh,
        out_type=jax.ShapeDtypeStruct((B, N_OUT, EMB), jnp.float32),
        compiler_params=pltpu.CompilerParams(
            needs_layout_passes=False, use_tc_tiling_on_sc=False),
        scratch_types=[
            pltpu.VMEM((2, TOK_W), jnp.int32),
            pltpu.VMEM((2, SEQ_W), jnp.int32),
            pltpu.VMEM((2, CB * N_FLOAT_FIELDS), jnp.float32),
            pltpu.VMEM((N_FLOAT_FIELDS, EMB), jnp.float32),
            pltpu.VMEM((TOK_W,), jnp.int32),
            pltpu.VMEM((2, TOK_W, EMB), jnp.float32),
            pltpu.VMEM((2, SEQ_W, EMB), jnp.float32),
            pltpu.VMEM((2, CB, N_OUT, EMB), jnp.float32),
            pltpu.SemaphoreType.DMA,
            pltpu.SemaphoreType.DMA,
            pltpu.SemaphoreType.DMA,
            pltpu.SemaphoreType.DMA,
            pltpu.SemaphoreType.DMA,
            pltpu.SemaphoreType.DMA,
        ],
    )
    def body(tok_idx_hbm, seq_idx_hbm, ff_hbm, tok_tab, seq_tab, ft_hbm,
             out_hbm, tok_idx_v, seq_idx_v, ff_v, ft_v, offs_v, tok_buf,
             seq_buf, out_v, sem_g0, sem_g1, sem_i0, sem_i1, sem_o0, sem_o1):
        nc = plsc.get_sparse_core_info().num_cores
        wid = lax.axis_index("s") * nc + lax.axis_index("c")
        sem_g = (sem_g0, sem_g1)
        sem_i = (sem_i0, sem_i1)
        sem_o = (sem_o0, sem_o1)

        def fire_idx(j, p):
            base = wid * b_per_w + j * CB
            pltpu.async_copy(
                tok_idx_hbm.at[pl.ds(base * N_TOKEN_FIELDS, TOK_W)],
                tok_idx_v.at[p], sem_i[p])
            pltpu.async_copy(
                seq_idx_hbm.at[pl.ds(base * SEQ_LEN, SEQ_W)],
                seq_idx_v.at[p], sem_i[p])
            pltpu.async_copy(
                ff_hbm.at[pl.ds(base * N_FLOAT_FIELDS, CB * N_FLOAT_FIELDS)],
                ff_v.at[p], sem_i[p])

        def wait_idx(p):
            pltpu.make_async_copy(tok_idx_hbm.at[pl.ds(0, TOK_W)],
                                  tok_idx_v.at[p], sem_i[p]).wait()
            pltpu.make_async_copy(seq_idx_hbm.at[pl.ds(0, SEQ_W)],
                                  seq_idx_v.at[p], sem_i[p]).wait()
            pltpu.make_async_copy(ff_hbm.at[pl.ds(0, CB * N_FLOAT_FIELDS)],
                                  ff_v.at[p], sem_i[p]).wait()

        def add_offsets(p):
            for g in range(TOK_W // 16):
                sl = pl.ds(g * 16, 16)
                tok_idx_v[p, sl] = tok_idx_v[p, sl] + offs_v[sl]

        def gather_descs(p):
            for g in range(4):
                yield pltpu.make_async_copy(
                    tok_tab.at[tok_idx_v.at[p, pl.ds(g * 104, 104)]],
                    tok_buf.at[p, pl.ds(g * 104, 104)], sem_g[p])
            for off, ln in SEQ_SEGS:
                yield pltpu.make_async_copy(
                    seq_tab.at[seq_idx_v.at[p, pl.ds(off, ln)]],
                    seq_buf.at[p, pl.ds(off, ln)], sem_g[p])

        def fire_gathers(p):
            for d in gather_descs(p):
                d.start()

        def drain_gathers(p):
            for d in gather_descs(p):
                d.wait()

        def fire_out(j, p):
            base = wid * b_per_w + j * CB
            pltpu.async_copy(out_v.at[p], out_hbm.at[pl.ds(base, CB)],
                             sem_o[p])

        def wait_out(p):
            pltpu.make_async_copy(out_v.at[p], out_hbm.at[pl.ds(0, CB)],
                                  sem_o[p]).wait()

        pltpu.sync_copy(ft_hbm, ft_v)
        ft_rows = [ft_v[f] for f in range(N_FLOAT_FIELDS)]
        lanes = jnp.arange(16, dtype=jnp.int32)
        lanes_seq = lanes * SEQ_LEN

        for g in range(TOK_W // 16):
            pvec = lanes + jnp.int32(g * 16)
            offs_v[pl.ds(g * 16, 16)] = (pvec % N_TOKEN_FIELDS) * FIELD_DIM

        fire_idx(0, 0)
        wait_idx(0)
        add_offsets(0)
        fire_gathers(0)
        fire_idx(1, 1)

        def pair(t, carry):
            for k in range(2):
                j = 2 * t + k
                p, q = k, 1 - k

                drain_gathers(p)

                cacc = jnp.zeros((16,), jnp.int32)
                for l in range(SEQ_LEN):
                    col = plsc.load_gather(seq_idx_v.at[p], [lanes_seq + l])
                    cacc = cacc + jnp.where(col != 0, 1, 0)
                inv_vec = jnp.float32(1.0) / (
                    cacc.astype(jnp.float32) + jnp.float32(1e-8))
                ffvecs = [ff_v[p, pl.ds(g * 16, 16)]
                          for g in range(CB * N_FLOAT_FIELDS // 16)]

                @pl.when(j + 2 < n_chunks)
                def _():
                    fire_idx(j + 2, p)

                @pl.when(j >= 1)
                def _():
                    wait_out(q)

                @pl.when(j + 1 < n_chunks)
                def _():
                    wait_idx(q)
                    add_offsets(q)
                    fire_gathers(q)

                for i in range(CB):
                    for c in range(N_TOKEN_FIELDS):
                        out_v[p, i, c] = tok_buf[p, i * N_TOKEN_FIELDS + c]
                    accs = [jnp.zeros((EMB,), jnp.float32) for _ in range(4)]
                    for l in range(SEQ_LEN):
                        accs[l % 4] = accs[l % 4] + seq_buf[p, i * SEQ_LEN + l]
                    summed = (accs[0] + accs[1]) + (accs[2] + accs[3])
                    out_v[p, i, N_TOKEN_FIELDS] = summed * inv_vec[i]
                    for f in range(N_FLOAT_FIELDS):
                        pos = i * N_FLOAT_FIELDS + f
                        val = ffvecs[pos // 16][pos % 16]
                        out_v[p, i, N_TOKEN_FIELDS + 1 + f] = ft_rows[f] * val

                fire_out(j, p)
            return carry

        lax.fori_loop(0, n_chunks // 2, pair, None)
        wait_out(1)

    return body


def kernel(token_fields, token_seq_field, float_fields, token_table,
           seq_table, float_table):
    info = plsc.get_sparse_core_info()
    nw = info.num_cores * info.num_subcores
    return _make_kernel(nw)(
        token_fields.astype(jnp.int32).reshape(B * N_TOKEN_FIELDS),
        token_seq_field.astype(jnp.int32).reshape(B * SEQ_LEN),
        float_fields.reshape(B * N_FLOAT_FIELDS),
        token_table, seq_table, float_table)


# speedup vs baseline: 5.1103x; 1.3241x over previous
import functools

import jax
import jax.numpy as jnp
from jax import lax
from jax.experimental import pallas as pl
from jax.experimental.pallas import tpu as pltpu
from jax.experimental.pallas import tpu_sc as plsc

B = 16384
N_TOKEN_FIELDS = 26
FIELD_DIM = 40000
SEQ_LEN = 50
N_FLOAT_FIELDS = 8
EMB = 16
N_OUT = N_TOKEN_FIELDS + 1 + N_FLOAT_FIELDS

CB = 16
TOK_W = CB * N_TOKEN_FIELDS
SEQ_W = CB * SEQ_LEN
SEQ_SEGS = [(0, 120), (120, 120), (240, 120), (360, 120),
            (480, 120), (600, 120), (720, 80)]


def _make_kernel(nw):
    b_per_w = B // nw
    n_chunks = b_per_w // CB
    mesh = plsc.VectorSubcoreMesh(core_axis_name="c", subcore_axis_name="s")

    @functools.partial(
        pl.kernel,
        mesh=mesh,
        out_type=jax.ShapeDtypeStruct((N_OUT, EMB, B), jnp.float32),
        compiler_params=pltpu.CompilerParams(
            needs_layout_passes=False, use_tc_tiling_on_sc=False),
        scratch_types=[
            pltpu.VMEM((2, TOK_W), jnp.int32),
            pltpu.VMEM((2, SEQ_W), jnp.int32),
            pltpu.VMEM((2, CB * N_FLOAT_FIELDS), jnp.float32),
            pltpu.VMEM((N_FLOAT_FIELDS, EMB), jnp.float32),
            pltpu.VMEM((TOK_W,), jnp.int32),
            pltpu.VMEM((2, TOK_W, EMB), jnp.float32),
            pltpu.VMEM((2, SEQ_W, EMB), jnp.float32),
            pltpu.VMEM((2, N_OUT, EMB, CB), jnp.float32),
            pltpu.SemaphoreType.DMA,
            pltpu.SemaphoreType.DMA,
            pltpu.SemaphoreType.DMA,
            pltpu.SemaphoreType.DMA,
            pltpu.SemaphoreType.DMA,
            pltpu.SemaphoreType.DMA,
        ],
    )
    def body(tok_idx_hbm, seq_idx_hbm, ff_hbm, tok_tab, seq_tab, ft_hbm,
             out_hbm, tok_idx_v, seq_idx_v, ff_v, ft_v, offs_v, tok_buf,
             seq_buf, out_v, sem_g0, sem_g1, sem_i0, sem_i1, sem_o0, sem_o1):
        nc = plsc.get_sparse_core_info().num_cores
        wid = lax.axis_index("s") * nc + lax.axis_index("c")
        sem_g = (sem_g0, sem_g1)
        sem_i = (sem_i0, sem_i1)
        sem_o = (sem_o0, sem_o1)

        def fire_idx(j, p):
            base = wid * b_per_w + j * CB
            pltpu.async_copy(
                tok_idx_hbm.at[pl.ds(base * N_TOKEN_FIELDS, TOK_W)],
                tok_idx_v.at[p], sem_i[p])
            pltpu.async_copy(
                seq_idx_hbm.at[pl.ds(base * SEQ_LEN, SEQ_W)],
                seq_idx_v.at[p], sem_i[p])
            pltpu.async_copy(
                ff_hbm.at[pl.ds(base * N_FLOAT_FIELDS, CB * N_FLOAT_FIELDS)],
                ff_v.at[p], sem_i[p])

        def wait_idx(p):
            pltpu.make_async_copy(tok_idx_hbm.at[pl.ds(0, TOK_W)],
                                  tok_idx_v.at[p], sem_i[p]).wait()
            pltpu.make_async_copy(seq_idx_hbm.at[pl.ds(0, SEQ_W)],
                                  seq_idx_v.at[p], sem_i[p]).wait()
            pltpu.make_async_copy(ff_hbm.at[pl.ds(0, CB * N_FLOAT_FIELDS)],
                                  ff_v.at[p], sem_i[p]).wait()

        def add_offsets(p):
            for g in range(TOK_W // 16):
                sl = pl.ds(g * 16, 16)
                tok_idx_v[p, sl] = tok_idx_v[p, sl] + offs_v[sl]

        def gather_descs(p):
            for g in range(4):
                yield pltpu.make_async_copy(
                    tok_tab.at[tok_idx_v.at[p, pl.ds(g * 104, 104)]],
                    tok_buf.at[p, pl.ds(g * 104, 104)], sem_g[p])
            for off, ln in SEQ_SEGS:
                yield pltpu.make_async_copy(
                    seq_tab.at[seq_idx_v.at[p, pl.ds(off, ln)]],
                    seq_buf.at[p, pl.ds(off, ln)], sem_g[p])

        def fire_gathers(p):
            for d in gather_descs(p):
                d.start()

        def drain_gathers(p):
            for d in gather_descs(p):
                d.wait()

        def fire_out(j, p):
            base = wid * b_per_w + j * CB
            pltpu.async_copy(out_v.at[p],
                             out_hbm.at[:, :, pl.ds(base, CB)], sem_o[p])

        def wait_out(p):
            pltpu.make_async_copy(out_v.at[p],
                                  out_hbm.at[:, :, pl.ds(0, CB)],
                                  sem_o[p]).wait()

        pltpu.sync_copy(ft_hbm, ft_v)
        ft_rows = [ft_v[f] for f in range(N_FLOAT_FIELDS)]
        lanes = jnp.arange(16, dtype=jnp.int32)
        lanes_seq = lanes * SEQ_LEN

        for g in range(TOK_W // 16):
            pvec = lanes + jnp.int32(g * 16)
            offs_v[pl.ds(g * 16, 16)] = (pvec % N_TOKEN_FIELDS) * FIELD_DIM

        fire_idx(0, 0)
        wait_idx(0)
        add_offsets(0)
        fire_gathers(0)
        fire_idx(1, 1)

        def pair(t, carry):
            for k in range(2):
                j = 2 * t + k
                p, q = k, 1 - k

                drain_gathers(p)

                cacc = jnp.zeros((16,), jnp.int32)
                for l in range(SEQ_LEN):
                    col = plsc.load_gather(seq_idx_v.at[p], [lanes_seq + l])
                    cacc = cacc + jnp.where(col != 0, 1, 0)
                inv_vec = jnp.float32(1.0) / (
                    cacc.astype(jnp.float32) + jnp.float32(1e-8))
                ffvecs = [ff_v[p, pl.ds(g * 16, 16)]
                          for g in range(CB * N_FLOAT_FIELDS // 16)]

                @pl.when(j + 2 < n_chunks)
                def _():
                    fire_idx(j + 2, p)

                @pl.when(j >= 1)
                def _():
                    wait_out(q)

                @pl.when(j + 1 < n_chunks)
                def _():
                    wait_idx(q)
                    add_offsets(q)
                    fire_gathers(q)

                for i in range(CB):
                    icol = jnp.full((16,), i, jnp.int32)
                    for c in range(N_TOKEN_FIELDS):
                        plsc.store_scatter(
                            out_v.at[p], [jnp.full((16,), c, jnp.int32),
                                          lanes, icol],
                            tok_buf[p, i * N_TOKEN_FIELDS + c])
                    accs = [jnp.zeros((EMB,), jnp.float32) for _ in range(4)]
                    for l in range(SEQ_LEN):
                        accs[l % 4] = accs[l % 4] + seq_buf[p, i * SEQ_LEN + l]
                    summed = (accs[0] + accs[1]) + (accs[2] + accs[3])
                    plsc.store_scatter(
                        out_v.at[p], [jnp.full((16,), N_TOKEN_FIELDS,
                                               jnp.int32), lanes, icol],
                        summed * inv_vec[i])
                    for f in range(N_FLOAT_FIELDS):
                        pos = i * N_FLOAT_FIELDS + f
                        val = ffvecs[pos // 16][pos % 16]
                        plsc.store_scatter(
                            out_v.at[p],
                            [jnp.full((16,), N_TOKEN_FIELDS + 1 + f,
                                      jnp.int32), lanes, icol],
                            ft_rows[f] * val)

                fire_out(j, p)
            return carry

        lax.fori_loop(0, n_chunks // 2, pair, None)
        wait_out(1)

    return body


def kernel(token_fields, token_seq_field, float_fields, token_table,
           seq_table, float_table):
    info = plsc.get_sparse_core_info()
    nw = info.num_cores * info.num_subcores
    out_t = _make_kernel(nw)(
        token_fields.astype(jnp.int32).reshape(B * N_TOKEN_FIELDS),
        token_seq_field.astype(jnp.int32).reshape(B * SEQ_LEN),
        float_fields.reshape(B * N_FLOAT_FIELDS),
        token_table, seq_table, float_table)
    return out_t.transpose(2, 0, 1)
